# baseline (speedup 1.0000x reference)
import jax
import jax.numpy as jnp
from jax import lax
from jax.experimental import pallas as pl
from jax.experimental.pallas import tpu as pltpu
from jax.experimental.pallas import tpu_sc as plsc

D_MODEL = 2048
N_GROUPS = 4
KEEP_RATE = 0.9
TOPK = 8
B_SZ, S_SZ = 2, 2048
T_TOKENS = B_SZ * S_SZ
TOK_TILE = 1024

NUM_WORKERS = 32
T_PER_W = S_SZ // NUM_WORKERS
L = 16


def _gates_body(x_ref, wa_ref, ba_ref, wk_ref, bk_ref, wta_ref, bta_ref,
                wtk_ref, btk_ref, emb_ref, eg_ref, embn_ref):
    @pl.when(pl.program_id(0) == 0)
    def _():
        emb = emb_ref[...]
        nrm = jnp.sqrt(jnp.sum(emb * emb, axis=1, keepdims=True))
        embn_ref[...] = emb / (nrm + 1e-8)

    emb_n = embn_ref[...]
    wt = jnp.concatenate([wta_ref[...], wtk_ref[...]], axis=1)
    bt = jnp.concatenate([bta_ref[...], btk_ref[...]])

    x = x_ref[...]
    h_attn = (jnp.dot(x, wa_ref[...], preferred_element_type=jnp.float32)
              + ba_ref[...]) / KEEP_RATE
    h_know = (jnp.dot(x, wk_ref[...], preferred_element_type=jnp.float32)
              + bk_ref[...]) / KEEP_RATE
    tau = (jnp.dot(x, wt, preferred_element_type=jnp.float32) + bt)

    outs = []
    for g in range(N_GROUPS):
        h_g = h_know if g == 3 else h_attn[:, 128 * g:128 * (g + 1)]
        if g <= 1:
            emb_g = emb_n[0:64]
        elif g == 2:
            emb_g = emb_n[64:128]
        else:
            emb_g = emb_n[128:192]
        scores = lax.dot_general(
            h_g, emb_g, (((1,), (1,)), ((), ())),
            preferred_element_type=jnp.float32)
        raw = scores - tau[:, g:g + 1]
        gate = jnp.where(raw > 0, raw, 1e-8 * jnp.exp(raw))
        outs.append(jnp.exp(gate) - 1.0)
    eg_ref[...] = jnp.concatenate(outs, axis=1)


def _compute_gates(xb, W_attn, b_attn, W_know, b_know,
                   W_tau_attn, b_tau_attn, W_tau_know, b_tau_know, emb):
    steps = S_SZ // TOK_TILE
    full = lambda shape: pl.BlockSpec(shape, lambda i: tuple(0 for _ in shape))
    return pl.pallas_call(
        _gates_body,
        grid=(steps,),
        in_specs=[
            pl.BlockSpec((TOK_TILE, D_MODEL), lambda i: (i, 0)),
            full((D_MODEL, 384)), full((384,)),
            full((D_MODEL, 128)), full((128,)),
            full((D_MODEL, 3)), full((3,)),
            full((D_MODEL, 1)), full((1,)),
            full((192, 128)),
        ],
        out_specs=pl.BlockSpec((TOK_TILE, 256), lambda i: (i, 0)),
        out_shape=jax.ShapeDtypeStruct((S_SZ, 256), jnp.float32),
        scratch_shapes=[pltpu.VMEM((192, 128), jnp.float32)],
    )(xb, W_attn, b_attn, W_know, b_know,
      W_tau_attn, b_tau_attn, W_tau_know, b_tau_know, emb)


def _gather16(x, idx):
    dn = lax.GatherDimensionNumbers(
        offset_dims=(), collapsed_slice_dims=(0,), start_index_map=(0,))
    return lax.gather(x, idx.reshape(L, 1), dn, (1,),
                      mode=lax.GatherScatterMode.PROMISE_IN_BOUNDS)


def _router_body(eg_hbm, v0, i0, v1, i1, v2, i2, v3, i3, part_hbm,
                 eg_v, sv_v, si_v, acc_v):
    wid = lax.axis_index("s") * 2 + lax.axis_index("c")
    base = wid * T_PER_W

    pltpu.sync_copy(eg_hbm.at[pl.ds(base, T_PER_W)], eg_v)

    zero16 = jnp.zeros((L,), jnp.float32)
    for i in range(256 // L):
        acc_v[pl.ds(L * i, L)] = zero16

    iota = lax.iota(jnp.int32, L)
    vidx = [iota + L * j for j in range(4)]
    nidx = [-1.0 - (iota + L * j).astype(jnp.float32) for j in range(4)]
    idx7 = jnp.full((L,), TOPK - 1, jnp.int32)
    idx0 = jnp.zeros((L,), jnp.int32)
    row8 = iota % 8
    lo8 = iota < 8

    def one_token(t, g):
        col0 = 64 * g
        e = [eg_v[t, pl.ds(col0 + L * j, L)] for j in range(4)]
        srt = [
            plsc.sort_key_val(jnp.where(e[j] > 0, e[j], nidx[j]), vidx[j],
                              descending=(j % 2 == 0))
            for j in range(4)
        ]

        def max_merge(a, b, descending):
            m = a[0] >= b[0]
            return plsc.sort_key_val(jnp.where(m, a[0], b[0]),
                                     jnp.where(m, a[1], b[1]),
                                     descending=descending)

        k01 = max_merge(srt[0], srt[1], True)
        k23 = max_merge(srt[2], srt[3], False)
        kt, vt = max_merge(k01, k23, True)

        thr = jnp.maximum(_gather16(kt, idx7), 0.0)
        mxv = jnp.maximum(_gather16(kt, idx0), 0.0)
        kept = [jnp.where(e[j] >= thr, e[j], 0.0) for j in range(4)]
        tot = lax.reduce_sum_p.bind(
            kept[0] + kept[1] + kept[2] + kept[3], axes=(0,))
        tot_b = lax.broadcast_in_dim(tot, (L,), ())
        ex = jnp.exp(2.0 * jnp.minimum(mxv, 20.0))
        strength = (ex - 1.0) / (ex + 1.0)
        scale = strength / (tot_b + 1e-8)

        for j in range(4):
            plsc.addupdate(acc_v.at[pl.ds(col0 + L * j, L)], kept[j] * scale)
        return jnp.maximum(kt, 0.0) * scale, vt

    @plsc.parallel_loop(0, T_PER_W, 1, unroll=2)
    def token_body(t):
        col = lax.broadcast_in_dim(t, (L,), ()).astype(jnp.int32)
        flat = row8 * T_PER_W + col
        fhi = lax.shift_right_logical(flat, 7)
        flo = lax.bitwise_and(flat, jnp.full((L,), 127, jnp.int32))
        for g in range(N_GROUPS):
            va, ia = one_token(t, g)
            plsc.store_scatter(sv_v.at[g], [fhi, flo], va, mask=lo8)
            plsc.store_scatter(si_v.at[g], [fhi, flo], ia, mask=lo8)

    for g, ref in enumerate([v0, v1, v2, v3]):
        pltpu.sync_copy(sv_v.at[g], ref.at[wid])
    for g, ref in enumerate([i0, i1, i2, i3]):
        pltpu.sync_copy(si_v.at[g], ref.at[wid])
    pltpu.sync_copy(acc_v, part_hbm.at[wid])


def _route(eg):
    mesh = plsc.VectorSubcoreMesh(core_axis_name="c", subcore_axis_name="s")
    tv = jax.ShapeDtypeStruct((NUM_WORKERS, TOPK * T_PER_W // 128, 128),
                              jnp.float32)
    ti = jax.ShapeDtypeStruct((NUM_WORKERS, TOPK * T_PER_W // 128, 128),
                              jnp.int32)
    return pl.kernel(
        _router_body,
        out_type=[tv, ti, tv, ti, tv, ti, tv, ti,
                  jax.ShapeDtypeStruct((NUM_WORKERS, 256), jnp.float32)],
        mesh=mesh,
        scratch_types=[
            pltpu.VMEM((T_PER_W, 256), jnp.float32),
            pltpu.VMEM((N_GROUPS, TOPK * T_PER_W // 128, 128), jnp.float32),
            pltpu.VMEM((N_GROUPS, TOPK * T_PER_W // 128, 128), jnp.int32),
            pltpu.VMEM((256,), jnp.float32),
        ],
        compiler_params=pltpu.CompilerParams(needs_layout_passes=False),
    )(eg)


def _aux_body(p0_ref, p1_ref, attn_ref, know_ref):
    p = p0_ref[...] + p1_ref[...]
    mean = jnp.sum(p, axis=0, keepdims=True) / T_TOKENS
    d = mean - (1.0 / 64.0)
    d2 = d * d
    attn_ref[...] = jnp.sum(d2[:, :192], keepdims=True) * 64.0
    know_ref[...] = jnp.sum(d2[:, 192:], keepdims=True) * 64.0


def _aux(p0, p1):
    return pl.pallas_call(
        _aux_body,
        out_shape=[jax.ShapeDtypeStruct((1, 1), jnp.float32),
                   jax.ShapeDtypeStruct((1, 1), jnp.float32)],
    )(p0, p1)


def kernel(x, neuron_emb, W_attn, b_attn, W_know, b_know,
           W_tau_attn, b_tau_attn, W_tau_know, b_tau_know):
    weights = (W_attn, b_attn, W_know, b_know,
               W_tau_attn, b_tau_attn, W_tau_know, b_tau_know, neuron_emb)
    eg0 = _compute_gates(x[0], *weights)
    r0 = _route(eg0)
    eg1 = _compute_gates(x[1], *weights)
    r1 = _route(eg1)
    aux_attn, aux_know = _aux(r0[8], r1[8])

    def tp(g):
        a = jnp.stack([r0[g], r1[g]])
        a = a.reshape(B_SZ, NUM_WORKERS, TOPK, T_PER_W)
        return a.transpose(0, 1, 3, 2).reshape(B_SZ, S_SZ, TOPK)

    return (tp(0), tp(1), tp(2), tp(3), tp(4), tp(5), aux_attn[0, 0],
            tp(6), tp(7), aux_know[0, 0])


# speedup vs baseline: 1.6689x; 1.6689x over previous
import jax
import jax.numpy as jnp
from jax import lax
from jax.experimental import pallas as pl
from jax.experimental.pallas import tpu as pltpu
from jax.experimental.pallas import tpu_sc as plsc

D_MODEL = 2048
N_GROUPS = 4
KEEP_RATE = 0.9
TOPK = 8
B_SZ, S_SZ = 2, 2048
T_TOKENS = B_SZ * S_SZ
TOK_TILE = 1024

NUM_WORKERS = 32
T_PER_W = T_TOKENS // NUM_WORKERS
W_PER_B = S_SZ // T_PER_W
L = 16


def _gates_body(x_ref, wa_ref, ba_ref, wk_ref, bk_ref, wta_ref, bta_ref,
                wtk_ref, btk_ref, emb_ref, eg_ref, embn_ref, wcat_ref):
    @pl.when(pl.program_id(0) == 0)
    def _():
        emb = emb_ref[...]
        nrm = jnp.sqrt(jnp.sum(emb * emb, axis=1, keepdims=True))
        embn_ref[...] = emb / (nrm + 1e-8)
        wcat_ref[:, 0:384] = wa_ref[...]
        wcat_ref[:, 384:512] = wk_ref[...]
        wcat_ref[:, 512:640] = jnp.concatenate(
            [wta_ref[...], wtk_ref[...],
             jnp.zeros((D_MODEL, 124), jnp.float32)], axis=1)

    emb_n = embn_ref[...]
    bt = jnp.concatenate([bta_ref[...], btk_ref[...]])

    x = x_ref[0]
    h = jnp.dot(x, wcat_ref[...], preferred_element_type=jnp.float32)
    h_attn = (h[:, 0:384] + ba_ref[...]) / KEEP_RATE
    h_know = (h[:, 384:512] + bk_ref[...]) / KEEP_RATE
    tau = h[:, 512:516] + bt

    outs = []
    for g in range(N_GROUPS):
        h_g = h_know if g == 3 else h_attn[:, 128 * g:128 * (g + 1)]
        if g <= 1:
            emb_g = emb_n[0:64]
        elif g == 2:
            emb_g = emb_n[64:128]
        else:
            emb_g = emb_n[128:192]
        scores = lax.dot_general(
            h_g, emb_g, (((1,), (1,)), ((), ())),
            preferred_element_type=jnp.float32)
        raw = scores - tau[:, g:g + 1]
        gate = jnp.where(raw > 0, raw, 1e-8 * jnp.exp(raw))
        outs.append(jnp.exp(gate) - 1.0)
    eg_ref[...] = jnp.concatenate(outs, axis=1)


def _compute_gates(x, W_attn, b_attn, W_know, b_know,
                   W_tau_attn, b_tau_attn, W_tau_know, b_tau_know, emb):
    steps = S_SZ // TOK_TILE
    full = lambda shape: pl.BlockSpec(shape, lambda i: tuple(0 for _ in shape))
    return pl.pallas_call(
        _gates_body,
        grid=(B_SZ * steps,),
        in_specs=[
            pl.BlockSpec((1, TOK_TILE, D_MODEL),
                         lambda i: (i // steps, i % steps, 0)),
            full((D_MODEL, 384)), full((384,)),
            full((D_MODEL, 128)), full((128,)),
            full((D_MODEL, 3)), full((3,)),
            full((D_MODEL, 1)), full((1,)),
            full((192, 128)),
        ],
        out_specs=pl.BlockSpec((TOK_TILE, 256), lambda i: (i, 0)),
        out_shape=jax.ShapeDtypeStruct((T_TOKENS, 256), jnp.float32),
        scratch_shapes=[pltpu.VMEM((192, 128), jnp.float32),
                        pltpu.VMEM((D_MODEL, 640), jnp.float32)],
    )(x, W_attn, b_attn, W_know, b_know,
      W_tau_attn, b_tau_attn, W_tau_know, b_tau_know, emb)


def _gather16(x, idx):
    dn = lax.GatherDimensionNumbers(
        offset_dims=(), collapsed_slice_dims=(0,), start_index_map=(0,))
    return lax.gather(x, idx.reshape(L, 1), dn, (1,),
                      mode=lax.GatherScatterMode.PROMISE_IN_BOUNDS)


def _router_body(eg_hbm, v0, i0, v1, i1, v2, i2, v3, i3, part_hbm,
                 eg_v, sv_v, si_v, acc_v):
    wid = lax.axis_index("s") * 2 + lax.axis_index("c")
    base = wid * T_PER_W

    pltpu.sync_copy(eg_hbm.at[pl.ds(base, T_PER_W)], eg_v)

    zero16 = jnp.zeros((L,), jnp.float32)
    for i in range(256 // L):
        acc_v[pl.ds(L * i, L)] = zero16

    iota = lax.iota(jnp.int32, L)
    vidx = [iota + L * j for j in range(4)]
    nidx = [-1.0 - (iota + L * j).astype(jnp.float32) for j in range(4)]
    idx7 = jnp.full((L,), TOPK - 1, jnp.int32)
    idx0 = jnp.zeros((L,), jnp.int32)
    row8 = iota % 8
    lo8 = iota < 8

    def one_token(t, g):
        col0 = 64 * g
        e = [eg_v[t, pl.ds(col0 + L * j, L)] for j in range(4)]
        srt = [
            plsc.sort_key_val(jnp.where(e[j] > 0, e[j], nidx[j]), vidx[j],
                              descending=(j % 2 == 0))
            for j in range(4)
        ]

        def max_merge(a, b, descending):
            m = a[0] >= b[0]
            return plsc.sort_key_val(jnp.where(m, a[0], b[0]),
                                     jnp.where(m, a[1], b[1]),
                                     descending=descending)

        k01 = max_merge(srt[0], srt[1], True)
        k23 = max_merge(srt[2], srt[3], False)
        kt, vt = max_merge(k01, k23, True)

        thr = jnp.maximum(_gather16(kt, idx7), 0.0)
        mxv = jnp.maximum(_gather16(kt, idx0), 0.0)
        kept = [jnp.where(e[j] >= thr, e[j], 0.0) for j in range(4)]
        tot = lax.reduce_sum_p.bind(
            kept[0] + kept[1] + kept[2] + kept[3], axes=(0,))
        tot_b = lax.broadcast_in_dim(tot, (L,), ())
        ex = jnp.exp(2.0 * jnp.minimum(mxv, 20.0))
        strength = (ex - 1.0) / (ex + 1.0)
        scale = strength / (tot_b + 1e-8)

        for j in range(4):
            plsc.addupdate(acc_v.at[pl.ds(col0 + L * j, L)], kept[j] * scale)
        return jnp.maximum(kt, 0.0) * scale, vt

    @plsc.parallel_loop(0, T_PER_W, 1, unroll=2)
    def token_body(t):
        col = lax.broadcast_in_dim(t, (L,), ()).astype(jnp.int32)
        for g in range(N_GROUPS):
            va, ia = one_token(t, g)
            plsc.store_scatter(sv_v.at[g], [row8, col], va, mask=lo8)
            plsc.store_scatter(si_v.at[g], [row8, col], ia, mask=lo8)

    b = wid // W_PER_B
    scol = (wid % W_PER_B) * T_PER_W
    for g, ref in enumerate([v0, v1, v2, v3]):
        pltpu.sync_copy(sv_v.at[g], ref.at[b, :, pl.ds(scol, T_PER_W)])
    for g, ref in enumerate([i0, i1, i2, i3]):
        pltpu.sync_copy(si_v.at[g], ref.at[b, :, pl.ds(scol, T_PER_W)])
    pltpu.sync_copy(acc_v, part_hbm.at[wid])


def _route(eg):
    mesh = plsc.VectorSubcoreMesh(core_axis_name="c", subcore_axis_name="s")
    tv = jax.ShapeDtypeStruct((B_SZ, TOPK, S_SZ), jnp.float32)
    ti = jax.ShapeDtypeStruct((B_SZ, TOPK, S_SZ), jnp.int32)
    return pl.kernel(
        _router_body,
        out_type=[tv, ti, tv, ti, tv, ti, tv, ti,
                  jax.ShapeDtypeStruct((NUM_WORKERS, 256), jnp.float32)],
        mesh=mesh,
        scratch_types=[
            pltpu.VMEM((T_PER_W, 256), jnp.float32),
            pltpu.VMEM((N_GROUPS, TOPK, T_PER_W), jnp.float32),
            pltpu.VMEM((N_GROUPS, TOPK, T_PER_W), jnp.int32),
            pltpu.VMEM((256,), jnp.float32),
        ],
        compiler_params=pltpu.CompilerParams(needs_layout_passes=False),
    )(eg)


def _aux_body(part_ref, attn_ref, know_ref):
    p = part_ref[...]
    mean = jnp.sum(p, axis=0, keepdims=True) / T_TOKENS
    d = mean - (1.0 / 64.0)
    d2 = d * d
    attn_ref[...] = jnp.sum(d2[:, :192], keepdims=True) * 64.0
    know_ref[...] = jnp.sum(d2[:, 192:], keepdims=True) * 64.0


def _aux(partials):
    return pl.pallas_call(
        _aux_body,
        out_shape=[jax.ShapeDtypeStruct((1, 1), jnp.float32),
                   jax.ShapeDtypeStruct((1, 1), jnp.float32)],
    )(partials)


def kernel(x, neuron_emb, W_attn, b_attn, W_know, b_know,
           W_tau_attn, b_tau_attn, W_tau_know, b_tau_know):
    eg = _compute_gates(x, W_attn, b_attn, W_know, b_know,
                        W_tau_attn, b_tau_attn, W_tau_know, b_tau_know,
                        neuron_emb)
    (v0, i0, v1, i1, v2, i2, v3, i3, partials) = _route(eg)
    aux_attn, aux_know = _aux(partials)

    def tp(a):
        return jnp.transpose(a, (0, 2, 1))

    return (tp(v0), tp(i0), tp(v1), tp(i1), tp(v2), tp(i2), aux_attn[0, 0],
            tp(v3), tp(i3), aux_know[0, 0])


# speedup vs baseline: 1.6701x; 1.0007x over previous
import jax
import jax.numpy as jnp
from jax import lax
from jax.experimental import pallas as pl
from jax.experimental.pallas import tpu as pltpu
from jax.experimental.pallas import tpu_sc as plsc

D_MODEL = 2048
N_GROUPS = 4
KEEP_RATE = 0.9
TOPK = 8
B_SZ, S_SZ = 2, 2048
T_TOKENS = B_SZ * S_SZ
TOK_TILE = 512

NUM_WORKERS = 32
T_PER_W = T_TOKENS // NUM_WORKERS
W_PER_B = S_SZ // T_PER_W
L = 16


def _gates_body(x_ref, wa_ref, ba_ref, wk_ref, bk_ref, wta_ref, bta_ref,
                wtk_ref, btk_ref, emb_ref, eg_ref, embn_ref, wcat_ref):
    @pl.when(pl.program_id(0) == 0)
    def _():
        emb = emb_ref[...]
        nrm = jnp.sqrt(jnp.sum(emb * emb, axis=1, keepdims=True))
        embn_ref[...] = emb / (nrm + 1e-8)
        wcat_ref[:, 0:384] = wa_ref[...]
        wcat_ref[:, 384:512] = wk_ref[...]
        wcat_ref[:, 512:640] = jnp.concatenate(
            [wta_ref[...], wtk_ref[...],
             jnp.zeros((D_MODEL, 124), jnp.float32)], axis=1)

    emb_n = embn_ref[...]
    bt = jnp.concatenate([bta_ref[...], btk_ref[...]])

    x = x_ref[0]
    h = jnp.dot(x, wcat_ref[...], preferred_element_type=jnp.float32)
    h_attn = (h[:, 0:384] + ba_ref[...]) / KEEP_RATE
    h_know = (h[:, 384:512] + bk_ref[...]) / KEEP_RATE
    tau = h[:, 512:516] + bt

    outs = []
    for g in range(N_GROUPS):
        h_g = h_know if g == 3 else h_attn[:, 128 * g:128 * (g + 1)]
        if g <= 1:
            emb_g = emb_n[0:64]
        elif g == 2:
            emb_g = emb_n[64:128]
        else:
            emb_g = emb_n[128:192]
        scores = lax.dot_general(
            h_g, emb_g, (((1,), (1,)), ((), ())),
            preferred_element_type=jnp.float32)
        raw = scores - tau[:, g:g + 1]
        gate = jnp.where(raw > 0, raw, 1e-8 * jnp.exp(raw))
        outs.append(jnp.exp(gate) - 1.0)
    eg_ref[...] = jnp.concatenate(outs, axis=1)


def _compute_gates(x, W_attn, b_attn, W_know, b_know,
                   W_tau_attn, b_tau_attn, W_tau_know, b_tau_know, emb):
    steps = S_SZ // TOK_TILE
    full = lambda shape: pl.BlockSpec(shape, lambda i: tuple(0 for _ in shape))
    return pl.pallas_call(
        _gates_body,
        grid=(B_SZ * steps,),
        in_specs=[
            pl.BlockSpec((1, TOK_TILE, D_MODEL),
                         lambda i: (i // steps, i % steps, 0)),
            full((D_MODEL, 384)), full((384,)),
            full((D_MODEL, 128)), full((128,)),
            full((D_MODEL, 3)), full((3,)),
            full((D_MODEL, 1)), full((1,)),
            full((192, 128)),
        ],
        out_specs=pl.BlockSpec((TOK_TILE, 256), lambda i: (i, 0)),
        out_shape=jax.ShapeDtypeStruct((T_TOKENS, 256), jnp.float32),
        scratch_shapes=[pltpu.VMEM((192, 128), jnp.float32),
                        pltpu.VMEM((D_MODEL, 640), jnp.float32)],
    )(x, W_attn, b_attn, W_know, b_know,
      W_tau_attn, b_tau_attn, W_tau_know, b_tau_know, emb)


def _gather16(x, idx):
    dn = lax.GatherDimensionNumbers(
        offset_dims=(), collapsed_slice_dims=(0,), start_index_map=(0,))
    return lax.gather(x, idx.reshape(L, 1), dn, (1,),
                      mode=lax.GatherScatterMode.PROMISE_IN_BOUNDS)


def _router_body(eg_hbm, v0, i0, v1, i1, v2, i2, v3, i3, part_hbm,
                 eg_v, sv_v, si_v, acc_v):
    wid = lax.axis_index("s") * 2 + lax.axis_index("c")
    base = wid * T_PER_W

    pltpu.sync_copy(eg_hbm.at[pl.ds(base, T_PER_W)], eg_v)

    zero16 = jnp.zeros((L,), jnp.float32)
    for i in range(256 // L):
        acc_v[pl.ds(L * i, L)] = zero16

    iota = lax.iota(jnp.int32, L)
    vidx = [iota + L * j for j in range(4)]
    nidx = [-1.0 - (iota + L * j).astype(jnp.float32) for j in range(4)]
    idx7 = jnp.full((L,), TOPK - 1, jnp.int32)
    idx0 = jnp.zeros((L,), jnp.int32)
    row8 = iota % 8
    lo8 = iota < 8

    def one_token(t, g):
        col0 = 64 * g
        e = [eg_v[t, pl.ds(col0 + L * j, L)] for j in range(4)]
        srt = [
            plsc.sort_key_val(jnp.where(e[j] > 0, e[j], nidx[j]), vidx[j],
                              descending=(j % 2 == 0))
            for j in range(4)
        ]

        def max_merge(a, b, descending):
            m = a[0] >= b[0]
            return plsc.sort_key_val(jnp.where(m, a[0], b[0]),
                                     jnp.where(m, a[1], b[1]),
                                     descending=descending)

        k01 = max_merge(srt[0], srt[1], True)
        k23 = max_merge(srt[2], srt[3], False)
        kt, vt = max_merge(k01, k23, True)

        thr = jnp.maximum(_gather16(kt, idx7), 0.0)
        mxv = jnp.maximum(_gather16(kt, idx0), 0.0)
        kept = [jnp.where(e[j] >= thr, e[j], 0.0) for j in range(4)]
        tot = lax.reduce_sum_p.bind(
            kept[0] + kept[1] + kept[2] + kept[3], axes=(0,))
        tot_b = lax.broadcast_in_dim(tot, (L,), ())
        ex = jnp.exp(2.0 * jnp.minimum(mxv, 20.0))
        strength = (ex - 1.0) / (ex + 1.0)
        scale = strength / (tot_b + 1e-8)

        for j in range(4):
            plsc.addupdate(acc_v.at[pl.ds(col0 + L * j, L)], kept[j] * scale)
        return jnp.maximum(kt, 0.0) * scale, vt

    @plsc.parallel_loop(0, T_PER_W, 1, unroll=2)
    def token_body(t):
        col = lax.broadcast_in_dim(t, (L,), ()).astype(jnp.int32)
        for g in range(N_GROUPS):
            va, ia = one_token(t, g)
            plsc.store_scatter(sv_v.at[g], [row8, col], va, mask=lo8)
            plsc.store_scatter(si_v.at[g], [row8, col], ia, mask=lo8)

    b = wid // W_PER_B
    scol = (wid % W_PER_B) * T_PER_W
    for g, ref in enumerate([v0, v1, v2, v3]):
        pltpu.sync_copy(sv_v.at[g], ref.at[b, :, pl.ds(scol, T_PER_W)])
    for g, ref in enumerate([i0, i1, i2, i3]):
        pltpu.sync_copy(si_v.at[g], ref.at[b, :, pl.ds(scol, T_PER_W)])
    pltpu.sync_copy(acc_v, part_hbm.at[wid])


def _route(eg):
    mesh = plsc.VectorSubcoreMesh(core_axis_name="c", subcore_axis_name="s")
    tv = jax.ShapeDtypeStruct((B_SZ, TOPK, S_SZ), jnp.float32)
    ti = jax.ShapeDtypeStruct((B_SZ, TOPK, S_SZ), jnp.int32)
    return pl.kernel(
        _router_body,
        out_type=[tv, ti, tv, ti, tv, ti, tv, ti,
                  jax.ShapeDtypeStruct((NUM_WORKERS, 256), jnp.float32)],
        mesh=mesh,
        scratch_types=[
            pltpu.VMEM((T_PER_W, 256), jnp.float32),
            pltpu.VMEM((N_GROUPS, TOPK, T_PER_W), jnp.float32),
            pltpu.VMEM((N_GROUPS, TOPK, T_PER_W), jnp.int32),
            pltpu.VMEM((256,), jnp.float32),
        ],
        compiler_params=pltpu.CompilerParams(needs_layout_passes=False),
    )(eg)


def _aux_body(part_ref, attn_ref, know_ref):
    p = part_ref[...]
    mean = jnp.sum(p, axis=0, keepdims=True) / T_TOKENS
    d = mean - (1.0 / 64.0)
    d2 = d * d
    attn_ref[...] = jnp.sum(d2[:, :192], keepdims=True) * 64.0
    know_ref[...] = jnp.sum(d2[:, 192:], keepdims=True) * 64.0


def _aux(partials):
    return pl.pallas_call(
        _aux_body,
        out_shape=[jax.ShapeDtypeStruct((1, 1), jnp.float32),
                   jax.ShapeDtypeStruct((1, 1), jnp.float32)],
    )(partials)


def kernel(x, neuron_emb, W_attn, b_attn, W_know, b_know,
           W_tau_attn, b_tau_attn, W_tau_know, b_tau_know):
    eg = _compute_gates(x, W_attn, b_attn, W_know, b_know,
                        W_tau_attn, b_tau_attn, W_tau_know, b_tau_know,
                        neuron_emb)
    (v0, i0, v1, i1, v2, i2, v3, i3, partials) = _route(eg)
    aux_attn, aux_know = _aux(partials)

    def tp(a):
        return jnp.transpose(a, (0, 2, 1))

    return (tp(v0), tp(i0), tp(v1), tp(i1), tp(v2), tp(i2), aux_attn[0, 0],
            tp(v3), tp(i3), aux_know[0, 0])
